# baseline (speedup 1.0000x reference)
import functools

import jax
import jax.numpy as jnp
from jax.experimental import pallas as pl
from jax.experimental.pallas import tpu as pltpu

_NX, _NY, _NZ = 128, 128, 16
_E = 128
_C = 256
_YZ = _NY * _NZ
_N_CAM = 6
_HW = 32 * 88
_XB = 8


def _prep_kernel(feats_ref, wqT_ref, wv_ref, bT_ref, coefT_ref, acc_ref):
    i = pl.program_id(0)

    @pl.when(i == 0)
    def _():
        acc_ref[...] = jnp.zeros_like(acc_ref)

    acc_ref[...] += jnp.sum(feats_ref[0], axis=-1, keepdims=True)

    @pl.when(i == _N_CAM - 1)
    def _():
        dT = jax.lax.dot_general(
            wv_ref[...], acc_ref[...],
            (((0,), (0,)), ((), ())),
            preferred_element_type=jnp.float32,
        ) * (1.0 / (_N_CAM * _HW))
        wqT = wqT_ref[...]
        axc = 0.4 * wqT[:, 0:1]
        ayc = 0.4 * wqT[:, 1:2]
        azc = 0.4 * wqT[:, 2:3]
        dcol = (dT + bT_ref[...]
                - 25.6 * (wqT[:, 0:1] + wqT[:, 1:2]) - 3.2 * wqT[:, 2:3])
        coefT_ref[:, 0:1] = axc
        coefT_ref[:, 1:2] = ayc
        coefT_ref[:, 2:3] = azc
        coefT_ref[:, 3:4] = dcol
        coefT_ref[:, 4:8] = jnp.zeros((_E, 4), jnp.float32)


def _fill_kernel(coefT_ref, prop_ref, out_ref):
    i = pl.program_id(0)
    coefT = coefT_ref[...]
    ax = coefT[:, 0:1].reshape(_E, 1, 1)
    ay = coefT[:, 1:2]
    az = coefT[:, 2:3]
    d = coefT[:, 3:4]
    yz = jax.lax.broadcasted_iota(jnp.int32, (_E, _YZ), 1)
    t = (ay * (yz // _NZ).astype(jnp.float32)
         + az * (yz % _NZ).astype(jnp.float32) + d)
    xg = (jax.lax.broadcasted_iota(jnp.int32, (_E, _XB, _YZ), 1)
          + i * _XB).astype(jnp.float32)
    val = ax * xg + t[:, None, :]
    keep = (prop_ref[...] > 0)[None, :, :]
    out_ref[...] = jnp.where(keep, val, 0.0)


@functools.partial(jax.jit, static_argnames=())
def kernel(mlvl_feats_0, proposal, W_q, W_v, b):
    feats = mlvl_feats_0.reshape(_N_CAM, _C, _HW)
    wqT = W_q.T
    bT = b.reshape(_E, 1)
    coefT = pl.pallas_call(
        _prep_kernel,
        grid=(_N_CAM,),
        in_specs=[
            pl.BlockSpec((1, _C, _HW), lambda i: (i, 0, 0)),
            pl.BlockSpec((_E, 3), lambda i: (0, 0)),
            pl.BlockSpec((_C, _E), lambda i: (0, 0)),
            pl.BlockSpec((_E, 1), lambda i: (0, 0)),
        ],
        out_specs=pl.BlockSpec((_E, 8), lambda i: (0, 0)),
        out_shape=jax.ShapeDtypeStruct((_E, 8), jnp.float32),
        scratch_shapes=[pltpu.VMEM((_C, 1), jnp.float32)],
    )(feats, wqT, W_v, bT)

    prop2d = proposal.reshape(_NX, _YZ)
    vol = pl.pallas_call(
        _fill_kernel,
        grid=(_NX // _XB,),
        in_specs=[
            pl.BlockSpec((_E, 8), lambda i: (0, 0)),
            pl.BlockSpec((_XB, _YZ), lambda i: (i, 0)),
        ],
        out_specs=pl.BlockSpec((_E, _XB, _YZ), lambda i: (0, i, 0)),
        out_shape=jax.ShapeDtypeStruct((_E, _NX, _YZ), jnp.float32),
    )(coefT, prop2d)
    return vol.reshape(1, _E, _NX, _NY, _NZ)
